# baseline (speedup 1.0000x reference)
import jax
import jax.numpy as jnp
from jax import lax
from jax.experimental import pallas as pl
from jax.experimental.pallas import tpu as pltpu
from jax.experimental.pallas import tpu_sc as plsc

_K = 8
_R = 32
_N = 32768
_L = 16
_S = 8
_W = _R // _S
_C = _N // _W
_RPC = _S // 2


def _sc_body(logits_hbm, g_hbm, out_hbm, l_v, acc_v, x_v, stg_v, part_v,
             shared, sem):
    c = lax.axis_index("c")
    s = lax.axis_index("s")
    row_local = s // _W
    chunk = s % _W
    row = c * _RPC + row_local
    col = chunk * _C

    pltpu.async_copy(logits_hbm.at[row, pl.ds(col, _C)], l_v, sem).wait()
    zero = jnp.zeros((_L,), jnp.float32)
    for k in range(_K):
        pltpu.async_copy(g_hbm.at[k, row, pl.ds(col, _C)], x_v, sem).wait()

        @plsc.parallel_loop(0, _C, 8 * _L, unroll=2,
                            carry=(zero,) * 8)
        def p1(j, accs):
            sls = [pl.ds(pl.multiple_of(j + t * _L, _L), _L) for t in range(8)]
            xs = [l_v[sl] + x_v[sl] for sl in sls]
            es = [jnp.exp(x) for x in xs]
            for t in range(8):
                x_v[sls[t]] = es[t]
            return tuple(accs[t] + es[t] for t in range(8))

        sacc = ((p1[0] + p1[1]) + (p1[2] + p1[3])) + \
               ((p1[4] + p1[5]) + (p1[6] + p1[7]))

        stg_v[:] = sacc
        pltpu.sync_copy(stg_v, shared.at[1 + k, c, s])
        plsc.subcore_barrier()
        pltpu.sync_copy(shared.at[1 + k, c, pl.ds(row_local * _W, _W)], part_v)
        tot = part_v[0]
        for t in range(1, _W):
            tot = tot + part_v[t]
        ssum = tot[0]
        for t in range(1, _L):
            ssum = ssum + tot[t]
        r = 1.0 / jnp.full((_L,), ssum, dtype=jnp.float32)

        if k == 0:
            @plsc.parallel_loop(0, _C, _L, unroll=8)
            def p2(j):
                sl = pl.ds(pl.multiple_of(j, _L), _L)
                acc_v[sl] = x_v[sl] * r
        else:
            @plsc.parallel_loop(0, _C, _L, unroll=8)
            def p2(j):
                sl = pl.ds(pl.multiple_of(j, _L), _L)
                acc_v[sl] = jnp.maximum(acc_v[sl], x_v[sl] * r)

    pltpu.async_copy(acc_v, out_hbm.at[row, pl.ds(col, _C)], sem).wait()


def _sc_part(logits, g):
    mesh = plsc.VectorSubcoreMesh(core_axis_name="c", subcore_axis_name="s")
    return pl.kernel(
        _sc_body,
        mesh=mesh,
        out_type=jax.ShapeDtypeStruct((_S, _N), jnp.float32),
        scratch_types=[
            pltpu.VMEM((_C,), jnp.float32),
            pltpu.VMEM((_C,), jnp.float32),
            pltpu.VMEM((_C,), jnp.float32),
            pltpu.VMEM((_L,), jnp.float32),
            pltpu.VMEM((_W, _L), jnp.float32),
            pltpu.VMEM_SHARED((_K + 1, 2, 16, _L), jnp.float32),
            pltpu.SemaphoreType.DMA,
        ],
    )(logits, g)


_BR = 8


def _tc_body(logits_ref, g_ref, out_ref):
    l = logits_ref[...]
    e = jnp.exp(l[None, :, :] + g_ref[...])
    s = jnp.sum(e, axis=2, keepdims=True)
    p = e * (1.0 / s)
    out_ref[...] = jnp.max(p, axis=0)


def _tc_part(logits, g):
    off = _S // _BR
    return pl.pallas_call(
        _tc_body,
        grid=((_R - _S) // _BR,),
        in_specs=[
            pl.BlockSpec((_BR, _N), lambda i: (i + off, 0)),
            pl.BlockSpec((_K, _BR, _N), lambda i: (0, i + off, 0)),
        ],
        out_specs=pl.BlockSpec((_BR, _N), lambda i: (i + off, 0)),
        out_shape=jax.ShapeDtypeStruct((_R, _N), jnp.float32),
    )(logits, g)


def kernel(logits, g):
    tc_out = _tc_part(logits, g)
    sc_out = _sc_part(logits, g)
    out = lax.dynamic_update_slice(tc_out, sc_out, (0, 0))
    return (out, out)


# speedup vs baseline: 1.2196x; 1.2196x over previous
import jax
import jax.numpy as jnp
from jax import lax
from jax.experimental import pallas as pl
from jax.experimental.pallas import tpu as pltpu
from jax.experimental.pallas import tpu_sc as plsc

_K = 8
_R = 32
_N = 32768
_L = 16
_S = 8
_W = _R // _S
_C = _N // _W
_RPC = _S // 2


def _sc_body(logits_hbm, g_hbm, out_hbm, red_hbm, l_v, acc_v, x_v, stg_v,
             part_v, sem_l, sem_g, sem_r):
    c = lax.axis_index("c")
    s = lax.axis_index("s")
    row_local = s // _W
    chunk = s % _W
    row = c * _RPC + row_local
    col = chunk * _C

    cp_l = pltpu.async_copy(logits_hbm.at[row, pl.ds(col, _C)], l_v, sem_l)
    cps = [
        pltpu.async_copy(g_hbm.at[k, row, pl.ds(col, _C)], x_v.at[k],
                         sem_g.at[k])
        for k in range(_K)
    ]
    cp_l.wait()

    zero = jnp.zeros((_L,), jnp.float32)
    saccs = []
    for k in range(_K):
        cps[k].wait()

        @plsc.parallel_loop(0, _C, 8 * _L, unroll=2, carry=(zero,) * 8)
        def p1(j, accs):
            sls = [pl.ds(pl.multiple_of(j + t * _L, _L), _L) for t in range(8)]
            xs = [l_v[sl] + x_v[k, sl] for sl in sls]
            es = [jnp.exp(x) for x in xs]
            for t in range(8):
                x_v[k, sls[t]] = es[t]
            return tuple(accs[t] + es[t] for t in range(8))

        saccs.append(((p1[0] + p1[1]) + (p1[2] + p1[3])) +
                     ((p1[4] + p1[5]) + (p1[6] + p1[7])))

    for k in range(_K):
        stg_v[k] = saccs[k]
    pltpu.async_copy(stg_v, red_hbm.at[c, s], sem_r).wait()
    plsc.subcore_barrier()
    pltpu.async_copy(
        red_hbm.at[c, pl.ds(row_local * _W, _W)], part_v, sem_r
    ).wait()

    rs = []
    for k in range(_K):
        tot = part_v[0, k]
        for t in range(1, _W):
            tot = tot + part_v[t, k]
        ssum = tot[0]
        for t in range(1, _L):
            ssum = ssum + tot[t]
        rs.append(1.0 / jnp.full((_L,), ssum, dtype=jnp.float32))

    @plsc.parallel_loop(0, _C, 2 * _L, unroll=4)
    def p2(j):
        for t in range(2):
            sl = pl.ds(pl.multiple_of(j + t * _L, _L), _L)
            ps = [x_v[k, sl] * rs[k] for k in range(_K)]
            m01 = jnp.maximum(ps[0], ps[1])
            m23 = jnp.maximum(ps[2], ps[3])
            m45 = jnp.maximum(ps[4], ps[5])
            m67 = jnp.maximum(ps[6], ps[7])
            acc_v[sl] = jnp.maximum(jnp.maximum(m01, m23),
                                    jnp.maximum(m45, m67))

    pltpu.async_copy(acc_v, out_hbm.at[row, pl.ds(col, _C)], sem_l).wait()


def _sc_part(logits, g):
    mesh = plsc.VectorSubcoreMesh(core_axis_name="c", subcore_axis_name="s")
    out, _ = pl.kernel(
        _sc_body,
        mesh=mesh,
        out_type=(
            jax.ShapeDtypeStruct((_S, _N), jnp.float32),
            jax.ShapeDtypeStruct((2, 16, _K, _L), jnp.float32),
        ),
        scratch_types=[
            pltpu.VMEM((_C,), jnp.float32),
            pltpu.VMEM((_C,), jnp.float32),
            pltpu.VMEM((_K, _C), jnp.float32),
            pltpu.VMEM((_K, _L), jnp.float32),
            pltpu.VMEM((_W, _K, _L), jnp.float32),
            pltpu.SemaphoreType.DMA,
            pltpu.SemaphoreType.DMA((_K,)),
            pltpu.SemaphoreType.DMA,
        ],
    )(logits, g)
    return out


_BR = 8


def _tc_body(logits_ref, g_ref, out_ref):
    l = logits_ref[...]
    e = jnp.exp(l[None, :, :] + g_ref[...])
    s = jnp.sum(e, axis=2, keepdims=True)
    p = e * (1.0 / s)
    out_ref[...] = jnp.max(p, axis=0)


def _tc_part(logits, g):
    off = _S // _BR
    return pl.pallas_call(
        _tc_body,
        grid=((_R - _S) // _BR,),
        in_specs=[
            pl.BlockSpec((_BR, _N), lambda i: (i + off, 0)),
            pl.BlockSpec((_K, _BR, _N), lambda i: (0, i + off, 0)),
        ],
        out_specs=pl.BlockSpec((_BR, _N), lambda i: (i + off, 0)),
        out_shape=jax.ShapeDtypeStruct((_R, _N), jnp.float32),
    )(logits, g)


def kernel(logits, g):
    tc_out = _tc_part(logits, g)
    sc_out = _sc_part(logits, g)
    out = lax.dynamic_update_slice(tc_out, sc_out, (0, 0))
    return (out, out)
